# speedup vs baseline: 470.5731x; 470.5731x over previous
import jax
import jax.numpy as jnp
import numpy as np
from jax.experimental import pallas as pl
from jax.experimental.pallas import tpu as pltpu


def _fused(x1_ref, w1a_ref, w1b_ref, b1a_ref, b1b_ref,
           wk_ref, bk_ref, wq_ref, bq_ref, wv_ref, bv_ref,
           wskip_ref, cb_ref, scale_ref, beta_ref,
           fcw_ref, fcb_ref, out_ref):
    x1 = x1_ref[...]
    f32 = jnp.float32
    xa = jnp.dot(x1, w1a_ref[...], preferred_element_type=f32) + b1a_ref[...]
    xb = jnp.dot(x1, w1b_ref[...], preferred_element_type=f32) + b1b_ref[...]
    x2 = jnp.maximum(jnp.maximum(xa, xb), 0.0)

    k = jnp.dot(x2, wk_ref[...], preferred_element_type=f32) + bk_ref[...]
    q = jnp.dot(x2, wq_ref[...], preferred_element_type=f32) + bq_ref[...]
    v = jnp.dot(x2, wv_ref[...], preferred_element_type=f32) + bv_ref[...]
    b = x1.shape[0]
    agg = float(b) * jax.nn.sigmoid(k + q) * v
    x3 = agg + jnp.dot(x2, wskip_ref[...], preferred_element_type=f32) + cb_ref[...]
    x3 = x3 * scale_ref[...] + beta_ref[...]

    x4 = jnp.maximum(x3[:, :8], x3[:, 8:])

    g = jax.lax.dot_general(x4, x4, (((1,), (1,)), ((), ())),
                            preferred_element_type=f32)
    att = jax.nn.sigmoid(g)
    att = att / jnp.sum(att, axis=1, keepdims=True)
    x5 = jnp.dot(att, x4, preferred_element_type=f32)
    x6 = x5 + x4
    out_ref[...] = jnp.dot(x6, fcw_ref[...], preferred_element_type=f32) + fcb_ref[...]


def kernel(x, train, W1, b1, Wk, bk, Wq, bq, Wv, bv, Wskip, conv_bias,
           bn_gamma, bn_beta, fc_W, fc_b):
    B = x.shape[0]
    d = x.shape[1] * x.shape[2]
    h = Wk.shape[0]
    x1 = x.reshape(B, d)

    perm = np.concatenate([np.arange(0, h, 2), np.arange(1, h, 2)])
    cols_a = 2 * perm
    cols_b = 2 * perm + 1
    W1a = W1[:, cols_a]
    W1b = W1[:, cols_b]
    b1a = b1[cols_a].reshape(1, h)
    b1b = b1[cols_b].reshape(1, h)

    Wk_p = Wk[perm][:, perm]
    Wq_p = Wq[perm][:, perm]
    Wv_p = Wv[perm][:, perm]
    Wsk_p = Wskip[perm][:, perm]
    bk_p = bk[perm].reshape(1, h)
    bq_p = bq[perm].reshape(1, h)
    bv_p = bv[perm].reshape(1, h)
    cb_p = conv_bias[perm].reshape(1, h)
    scale = (bn_gamma / jnp.sqrt(1.0 + 1e-5))[perm].reshape(1, h)
    beta = bn_beta[perm].reshape(1, h)
    fcb = fc_b.reshape(1, fc_b.shape[0])

    out = pl.pallas_call(
        _fused,
        out_shape=jax.ShapeDtypeStruct((B, fc_W.shape[1]), jnp.float32),
    )(x1, W1a, W1b, b1a, b1b, Wk_p, bk_p, Wq_p, bq_p, Wv_p, bv_p,
      Wsk_p, cb_p, scale, beta, fc_W, fcb)
    return out


# speedup vs baseline: 575.3816x; 1.2227x over previous
import jax
import jax.numpy as jnp
import numpy as np
from jax.experimental import pallas as pl
from jax.experimental.pallas import tpu as pltpu


def _fused(x1_ref, w1_ref, b1_ref, wkqvs_ref, bkqvs_ref,
           cb_ref, scale_ref, beta_ref, fcw_ref, fcb_ref, out_ref):
    x1 = x1_ref[...]
    f32 = jnp.float32
    h = cb_ref.shape[1]
    xab = jnp.dot(x1, w1_ref[...], preferred_element_type=f32) + b1_ref[...]
    x2 = jnp.maximum(jnp.maximum(xab[:, :h], xab[:, h:]), 0.0)

    kqvs = jnp.dot(x2, wkqvs_ref[...], preferred_element_type=f32) + bkqvs_ref[...]
    k = kqvs[:, :h]
    q = kqvs[:, h:2 * h]
    v = kqvs[:, 2 * h:3 * h]
    skip = kqvs[:, 3 * h:]
    b = x1.shape[0]
    agg = float(b) * jax.nn.sigmoid(k + q) * v
    x3 = agg + skip + cb_ref[...]
    x3 = x3 * scale_ref[...] + beta_ref[...]

    hh = h // 2
    x4 = jnp.maximum(x3[:, :hh], x3[:, hh:])
    x4e = jnp.concatenate([x4, jnp.ones((b, 1), f32)], axis=1)

    g = jax.lax.dot_general(x4, x4, (((1,), (1,)), ((), ())),
                            preferred_element_type=f32)
    att = jax.nn.sigmoid(g)
    r = jnp.dot(att, x4e, preferred_element_type=f32)
    x6 = r[:, :hh] / r[:, hh:hh + 1] + x4
    out_ref[...] = jnp.dot(x6, fcw_ref[...], preferred_element_type=f32) + fcb_ref[...]


def kernel(x, train, W1, b1, Wk, bk, Wq, bq, Wv, bv, Wskip, conv_bias,
           bn_gamma, bn_beta, fc_W, fc_b):
    B = x.shape[0]
    d = x.shape[1] * x.shape[2]
    h = Wk.shape[0]
    x1 = x.reshape(B, d)

    perm = np.concatenate([np.arange(0, h, 2), np.arange(1, h, 2)])
    cols_a = 2 * perm
    cols_b = 2 * perm + 1
    W1cat = jnp.concatenate([W1[:, cols_a], W1[:, cols_b]], axis=1)
    b1cat = jnp.concatenate([b1[cols_a], b1[cols_b]]).reshape(1, 2 * h)

    Wkqvs = jnp.concatenate(
        [Wk[perm][:, perm], Wq[perm][:, perm], Wv[perm][:, perm],
         Wskip[perm][:, perm]], axis=1)
    bkqvs = jnp.concatenate(
        [bk[perm], bq[perm], bv[perm], jnp.zeros((h,), jnp.float32)]
    ).reshape(1, 4 * h)
    cb_p = conv_bias[perm].reshape(1, h)
    scale = (bn_gamma / jnp.sqrt(1.0 + 1e-5))[perm].reshape(1, h)
    beta = bn_beta[perm].reshape(1, h)
    fcb = fc_b.reshape(1, fc_b.shape[0])

    out = pl.pallas_call(
        _fused,
        out_shape=jax.ShapeDtypeStruct((B, fc_W.shape[1]), jnp.float32),
    )(x1, W1cat, b1cat, Wkqvs, bkqvs, cb_p, scale, beta, fc_W, fcb)
    return out


# speedup vs baseline: 846.2714x; 1.4708x over previous
import jax
import jax.numpy as jnp
import numpy as np
from jax.experimental import pallas as pl
from jax.experimental.pallas import tpu as pltpu

_H = 16


def _selectors(n):
    ri = jax.lax.broadcasted_iota(jnp.int32, (2 * n, n), 0)
    ci = jax.lax.broadcasted_iota(jnp.int32, (2 * n, n), 1)
    se = (ri == 2 * ci).astype(jnp.float32)
    so = (ri == 2 * ci + 1).astype(jnp.float32)
    return se, so


def _fused(x1_ref, w1_ref, b1_ref, wk_ref, bk_ref, wq_ref, bq_ref,
           wv_ref, bv_ref, wskip_ref, cb_ref, gamma_ref, beta_ref,
           fcw_ref, fcb_ref, out_ref):
    f32 = jnp.float32
    dot = lambda a, b: jnp.dot(a, b, preferred_element_type=f32)
    se32, so32 = _selectors(_H)
    se16, so16 = _selectors(_H // 2)

    x1 = x1_ref[...]
    xab = dot(x1, w1_ref[...]) + b1_ref[...]
    x2 = jnp.maximum(jnp.maximum(dot(xab, se32), dot(xab, so32)), 0.0)

    k = dot(x2, wk_ref[...]) + bk_ref[...]
    q = dot(x2, wq_ref[...]) + bq_ref[...]
    v = dot(x2, wv_ref[...]) + bv_ref[...]
    b = x1.shape[0]
    agg = float(b) * jax.nn.sigmoid(k + q) * v
    x3 = agg + dot(x2, wskip_ref[...]) + cb_ref[...]
    x3 = x3 * (gamma_ref[...] * (1.0 / np.sqrt(1.0 + 1e-5))) + beta_ref[...]

    x4 = jnp.maximum(dot(x3, se16), dot(x3, so16))
    x4e = jnp.concatenate([x4, jnp.ones((b, 1), f32)], axis=1)

    g = jax.lax.dot_general(x4, x4, (((1,), (1,)), ((), ())),
                            preferred_element_type=f32)
    att = jax.nn.sigmoid(g)
    r = dot(att, x4e)
    hh = _H // 2
    x6 = r[:, :hh] / r[:, hh:hh + 1] + x4
    out_ref[...] = dot(x6, fcw_ref[...]) + fcb_ref[...]


def kernel(x, train, W1, b1, Wk, bk, Wq, bq, Wv, bv, Wskip, conv_bias,
           bn_gamma, bn_beta, fc_W, fc_b):
    B = x.shape[0]
    d = x.shape[1] * x.shape[2]
    h = Wk.shape[0]
    x1 = x.reshape(B, d)
    row = lambda t: t.reshape(1, t.shape[0])

    out = pl.pallas_call(
        _fused,
        out_shape=jax.ShapeDtypeStruct((B, fc_W.shape[1]), jnp.float32),
    )(x1, W1, row(b1), Wk, row(bk), Wq, row(bq), Wv, row(bv), Wskip,
      row(conv_bias), row(bn_gamma), row(bn_beta), fc_W, row(fc_b))
    return out
